# baseline (speedup 1.0000x reference)
import functools

import jax
import jax.numpy as jnp
from jax import lax
from jax.experimental import pallas as pl
from jax.experimental.pallas import tpu as pltpu
from jax.experimental.pallas import tpu_sc as plsc

NUM_EXPERTS = 8
HIDDEN = 1024
INTER = 768
TOP_K = 2
T = 2048
A = T * TOP_K
BLK = 128
NB = 40
L = NB * BLK
NBPAD = 48


def _router_body(x_ref, gwt_ref, topi_ref, topw_ref):
    logits = lax.dot_general(
        x_ref[...], gwt_ref[...], (((1,), (0,)), ((), ())),
        precision=lax.Precision.DEFAULT,
        preferred_element_type=jnp.float32)
    m = jnp.max(logits, axis=-1, keepdims=True)
    p = jnp.exp(logits - m)
    w = p / jnp.sum(p, axis=-1, keepdims=True)
    iota = lax.broadcasted_iota(jnp.int32, w.shape, 1)
    w1 = jnp.max(w, axis=-1, keepdims=True)
    i1 = jnp.min(jnp.where(w == w1, iota, NUM_EXPERTS), axis=-1, keepdims=True)
    wm = jnp.where(iota == i1, -jnp.inf, w)
    w2 = jnp.max(wm, axis=-1, keepdims=True)
    i2 = jnp.min(jnp.where(wm == w2, iota, NUM_EXPERTS), axis=-1, keepdims=True)
    topi_ref[...] = jnp.concatenate([i1, i2], axis=1)
    topw_ref[...] = jnp.concatenate([w1, w2], axis=1)


def _router(x, gwt):
    return pl.pallas_call(
        _router_body,
        out_shape=(jax.ShapeDtypeStruct((T, TOP_K), jnp.int32),
                   jax.ShapeDtypeStruct((T, TOP_K), jnp.float32)),
    )(x, gwt)


_SORT_MESH = plsc.VectorSubcoreMesh(core_axis_name="c", subcore_axis_name="s")
_NSUB = 16
_CHUNK = A // _NSUB
_LSLICE = L // _NSUB


def _lane_iota():
    return lax.iota(jnp.int32, 16)


@functools.partial(
    pl.kernel, mesh=_SORT_MESH,
    out_type=(jax.ShapeDtypeStruct((L,), jnp.float32),
              jax.ShapeDtypeStruct((NBPAD,), jnp.int32),
              jax.ShapeDtypeStruct((T,), jnp.int32),
              jax.ShapeDtypeStruct((T,), jnp.int32)),
    scratch_types=[
        pltpu.VMEM((_CHUNK,), jnp.int32),
        pltpu.VMEM((_CHUNK,), jnp.float32),
        pltpu.VMEM((_CHUNK,), jnp.int32),
        pltpu.VMEM((_CHUNK // 2,), jnp.int32),
        pltpu.VMEM((_CHUNK // 2,), jnp.int32),
        pltpu.VMEM((16,), jnp.int32),
        pltpu.VMEM((16 * _NSUB,), jnp.int32),
        pltpu.VMEM((_LSLICE,), jnp.float32),
        pltpu.VMEM((NBPAD,), jnp.int32),
        pltpu.VMEM_SHARED((16 * _NSUB,), jnp.int32),
        pltpu.VMEM_SHARED((L,), jnp.float32),
    ],
)
def _sc_sort(topi_hbm, topw_hbm, sw_hbm, be_hbm, inv0_hbm, inv1_hbm,
             ti_v, tw_v, pos_v, pe_v, po_v, cnt_v, grid_v, zf_v, be_v,
             cnt_sh, sw_sh):
    cid = lax.axis_index("c")
    sid = lax.axis_index("s")

    @pl.when(cid == 0)
    def _():
        base_j = sid * _CHUNK
        lanes = _lane_iota()
        zvec = jnp.zeros((16,), jnp.int32)
        pltpu.sync_copy(topi_hbm.at[pl.ds(base_j, _CHUNK)], ti_v)
        pltpu.sync_copy(topw_hbm.at[pl.ds(base_j, _CHUNK)], tw_v)

        cvecs = [jnp.zeros((16,), jnp.int32) for _ in range(NUM_EXPERTS)]
        for v in range(_CHUNK // 16):
            ev = ti_v[pl.ds(v * 16, 16)]
            for e in range(NUM_EXPERTS):
                cvecs[e] = cvecs[e] + jnp.where(ev == e, 1, 0)
        counts = zvec
        for e in range(NUM_EXPERTS):
            c = cvecs[e]
            s = c[0]
            for i in range(1, 16):
                s = s + c[i]
            counts = jnp.where(lanes == e, zvec + s, counts)
        cnt_v[...] = counts
        pltpu.sync_copy(cnt_v, cnt_sh.at[pl.ds(sid * 16, 16)])

        for v in range(_LSLICE // 16):
            zf_v[pl.ds(v * 16, 16)] = jnp.zeros((16,), jnp.float32)
        pltpu.sync_copy(zf_v, sw_sh.at[pl.ds(sid * _LSLICE, _LSLICE)])
        plsc.subcore_barrier()

        pltpu.sync_copy(cnt_sh, grid_v)
        tot = [jnp.int32(0)] * NUM_EXPERTS
        soff = [jnp.int32(0)] * NUM_EXPERTS
        for r in range(_NSUB):
            row = grid_v[pl.ds(r * 16, 16)]
            mr = jnp.where(r < sid, 1, 0)
            for e in range(NUM_EXPERTS):
                re = row[e]
                tot[e] = tot[e] + re
                soff[e] = soff[e] + re * mr
        pad = [jnp.bitwise_and(t + (BLK - 1), -BLK) for t in tot]
        base = [0] * NUM_EXPERTS
        acc = pad[0] * 0
        for e in range(NUM_EXPERTS):
            base[e] = acc
            acc = acc + pad[e]
        start = [base[e] + soff[e] for e in range(NUM_EXPERTS)]

        @pl.when(sid == 0)
        def _():
            es = [zvec] * (NBPAD // 16)
            for e in range(NUM_EXPERTS):
                endb = zvec + (base[e] + pad[e]) // BLK
                for g in range(NBPAD // 16):
                    es[g] = es[g] + jnp.where(lanes + g * 16 >= endb, 1, 0)
            for g in range(NBPAD // 16):
                be_v[pl.ds(g * 16, 16)] = jnp.minimum(es[g], NUM_EXPERTS - 1)
            pltpu.sync_copy(be_v, be_hbm)

        def step(v, runs):
            ev = ti_v[pl.ds(v * 16, 16)]
            pos = zvec
            for i in range(16):
                e = ev[i]
                ms = [1 - jnp.minimum(jnp.abs(e - k), 1)
                      for k in range(NUM_EXPERTS)]
                p = runs[0] * ms[0]
                for k in range(1, NUM_EXPERTS):
                    p = p + runs[k] * ms[k]
                pos = jnp.where(lanes == i, zvec + p, pos)
                runs = tuple(runs[k] + ms[k] for k in range(NUM_EXPERTS))
            pos_v[pl.ds(v * 16, 16)] = pos
            return runs

        lax.fori_loop(0, _CHUNK // 16, step,
                      tuple(jnp.int32(0) + s for s in start))
        for g in range(_CHUNK // 32):
            v0 = pos_v[pl.ds(g * 32, 16)]
            v1 = pos_v[pl.ds(g * 32 + 16, 16)]
            pe = zvec
            po = zvec
            for i in range(8):
                pe = jnp.where(lanes == i, zvec + v0[2 * i], pe)
                pe = jnp.where(lanes == i + 8, zvec + v1[2 * i], pe)
                po = jnp.where(lanes == i, zvec + v0[2 * i + 1], po)
                po = jnp.where(lanes == i + 8, zvec + v1[2 * i + 1], po)
            pe_v[pl.ds(g * 16, 16)] = pe
            po_v[pl.ds(g * 16, 16)] = po
        tbase = sid * (_CHUNK // 2)
        pltpu.sync_copy(pe_v, inv0_hbm.at[pl.ds(tbase, _CHUNK // 2)])
        pltpu.sync_copy(po_v, inv1_hbm.at[pl.ds(tbase, _CHUNK // 2)])
        pltpu.sync_copy(tw_v, sw_sh.at[pos_v], add=True)
        plsc.subcore_barrier()

        pltpu.sync_copy(sw_sh.at[pl.ds(sid * _LSLICE, _LSLICE)], zf_v)
        pltpu.sync_copy(zf_v, sw_hbm.at[pl.ds(sid * _LSLICE, _LSLICE)])


_GMESH = plsc.VectorSubcoreMesh(core_axis_name="c", subcore_axis_name="s")
_NW = 32
_SROWS = T // _NW


@functools.partial(
    pl.kernel, mesh=_GMESH,
    out_type=jax.ShapeDtypeStruct((L, HIDDEN), jnp.float32),
    scratch_types=[
        pltpu.VMEM((_SROWS,), jnp.int32),
        pltpu.VMEM((_SROWS,), jnp.int32),
        pltpu.VMEM((_SROWS, HIDDEN), jnp.float32),
        pltpu.SemaphoreType.DMA,
        pltpu.SemaphoreType.DMA,
    ],
)
def _sc_scatter(x_hbm, inv0_hbm, inv1_hbm, xs_hbm, i0_v, i1_v, xr_v,
                sem0, sem1):
    wid = lax.axis_index("s") * 2 + lax.axis_index("c")
    base = wid * _SROWS
    pltpu.sync_copy(inv0_hbm.at[pl.ds(base, _SROWS)], i0_v)
    pltpu.sync_copy(inv1_hbm.at[pl.ds(base, _SROWS)], i1_v)
    pltpu.sync_copy(x_hbm.at[pl.ds(base, _SROWS)], xr_v)
    cp0 = pltpu.async_copy(xr_v, xs_hbm.at[i0_v], sem0)
    cp1 = pltpu.async_copy(xr_v, xs_hbm.at[i1_v], sem1)
    cp0.wait()
    cp1.wait()


def _experts_body(be_ref, xs_ref, gup_ref, dnt_ref, sw_ref, out_ref):
    xb = xs_ref[...].astype(jnp.bfloat16)
    gu = lax.dot_general(
        xb, gup_ref[0], (((1,), (0,)), ((), ())),
        preferred_element_type=jnp.float32)
    g = gu[:, :INTER]
    u = gu[:, INTER:]
    h = (g * (1.0 / (1.0 + jnp.exp(-g))) * u).astype(jnp.bfloat16)
    oe = lax.dot_general(
        h, dnt_ref[0], (((1,), (0,)), ((), ())),
        preferred_element_type=jnp.float32)
    out_ref[...] = oe * sw_ref[...]


def _experts(be, xs, gup_t, dnt, sw):
    grid_spec = pltpu.PrefetchScalarGridSpec(
        num_scalar_prefetch=1,
        grid=(NB,),
        in_specs=[
            pl.BlockSpec((BLK, HIDDEN), lambda b, be_ref: (b, 0)),
            pl.BlockSpec((1, HIDDEN, 2 * INTER),
                         lambda b, be_ref: (be_ref[b], 0, 0)),
            pl.BlockSpec((1, INTER, HIDDEN),
                         lambda b, be_ref: (be_ref[b], 0, 0)),
            pl.BlockSpec((BLK, 1), lambda b, be_ref: (b, 0)),
        ],
        out_specs=pl.BlockSpec((BLK, HIDDEN), lambda b, be_ref: (b, 0)),
    )
    return pl.pallas_call(
        _experts_body,
        grid_spec=grid_spec,
        out_shape=jax.ShapeDtypeStruct((L, HIDDEN), jnp.float32),
    )(be, xs, gup_t, dnt, sw)


_CTOK = T // _NW
_CCH = 16


@functools.partial(
    pl.kernel, mesh=_GMESH,
    out_type=jax.ShapeDtypeStruct((T, HIDDEN), jnp.float32),
    scratch_types=[
        pltpu.VMEM((_CTOK,), jnp.int32),
        pltpu.VMEM((_CTOK,), jnp.int32),
        pltpu.VMEM((_CCH, HIDDEN), jnp.float32),
        pltpu.VMEM((_CCH, HIDDEN), jnp.float32),
        pltpu.VMEM((_CCH, HIDDEN), jnp.float32),
        pltpu.SemaphoreType.DMA,
        pltpu.SemaphoreType.DMA,
    ],
)
def _sc_combine(os_hbm, inv0_hbm, inv1_hbm, out_hbm, i0_v, i1_v,
                g0_v, g1_v, o_v, sem0, sem1):
    wid = lax.axis_index("s") * 2 + lax.axis_index("c")
    tbase = wid * _CTOK
    pltpu.sync_copy(inv0_hbm.at[pl.ds(tbase, _CTOK)], i0_v)
    pltpu.sync_copy(inv1_hbm.at[pl.ds(tbase, _CTOK)], i1_v)
    for c in range(_CTOK // _CCH):
        cp0 = pltpu.async_copy(
            os_hbm.at[i0_v.at[pl.ds(c * _CCH, _CCH)]], g0_v, sem0)
        cp1 = pltpu.async_copy(
            os_hbm.at[i1_v.at[pl.ds(c * _CCH, _CCH)]], g1_v, sem1)
        cp0.wait()
        cp1.wait()

        def body(q, _):
            for i in range(_CCH):
                a = g0_v[i, pl.ds(q * 16, 16)]
                b = g1_v[i, pl.ds(q * 16, 16)]
                o_v[i, pl.ds(q * 16, 16)] = a + b
            return 0

        lax.fori_loop(0, HIDDEN // 16, body, 0)
        pltpu.sync_copy(o_v, out_hbm.at[pl.ds(tbase + c * _CCH, _CCH)])


def kernel(hidden_states, gate_weight, gate_up_proj, down_proj):
    Bb, Ss, H = hidden_states.shape
    x = hidden_states.reshape(T, H)
    gwt = gate_weight.T
    gup_t = gate_up_proj.transpose(0, 2, 1).astype(jnp.bfloat16)
    dnt = down_proj.transpose(0, 2, 1).astype(jnp.bfloat16)

    topi, topw = _router(x, gwt)
    topi_flat = topi.reshape(A)
    topw_flat = topw.reshape(A)
    sw, be, inv0, inv1 = _sc_sort(topi_flat, topw_flat)
    xs = _sc_scatter(x, inv0, inv1)
    out_sorted = _experts(be, xs, gup_t, dnt, sw.reshape(L, 1))
    out = _sc_combine(out_sorted, inv0, inv1)
    return out.reshape(Bb, Ss, H)


# speedup vs baseline: 1.0459x; 1.0459x over previous
import functools

import jax
import jax.numpy as jnp
from jax import lax
from jax.experimental import pallas as pl
from jax.experimental.pallas import tpu as pltpu
from jax.experimental.pallas import tpu_sc as plsc

NUM_EXPERTS = 8
HIDDEN = 1024
INTER = 768
TOP_K = 2
T = 2048
A = T * TOP_K
BLK = 256
NB = 24
L = NB * BLK
NBPAD = 32


def _router_body(x_ref, gwt_ref, topi_ref, topw_ref):
    logits = lax.dot_general(
        x_ref[...], gwt_ref[...], (((1,), (0,)), ((), ())),
        precision=lax.Precision.DEFAULT,
        preferred_element_type=jnp.float32)
    m = jnp.max(logits, axis=-1, keepdims=True)
    p = jnp.exp(logits - m)
    w = p / jnp.sum(p, axis=-1, keepdims=True)
    iota = lax.broadcasted_iota(jnp.int32, w.shape, 1)
    w1 = jnp.max(w, axis=-1, keepdims=True)
    i1 = jnp.min(jnp.where(w == w1, iota, NUM_EXPERTS), axis=-1, keepdims=True)
    wm = jnp.where(iota == i1, -jnp.inf, w)
    w2 = jnp.max(wm, axis=-1, keepdims=True)
    i2 = jnp.min(jnp.where(wm == w2, iota, NUM_EXPERTS), axis=-1, keepdims=True)
    topi_ref[...] = jnp.concatenate([i1, i2], axis=1)
    topw_ref[...] = jnp.concatenate([w1, w2], axis=1)


def _router(x, gwt):
    return pl.pallas_call(
        _router_body,
        out_shape=(jax.ShapeDtypeStruct((T, TOP_K), jnp.int32),
                   jax.ShapeDtypeStruct((T, TOP_K), jnp.float32)),
    )(x, gwt)


_SORT_MESH = plsc.VectorSubcoreMesh(core_axis_name="c", subcore_axis_name="s")
_NSUB = 16
_CHUNK = A // _NSUB
_LSLICE = L // _NSUB


def _lane_iota():
    return lax.iota(jnp.int32, 16)


@functools.partial(
    pl.kernel, mesh=_SORT_MESH,
    out_type=(jax.ShapeDtypeStruct((L,), jnp.float32),
              jax.ShapeDtypeStruct((NBPAD,), jnp.int32),
              jax.ShapeDtypeStruct((T,), jnp.int32),
              jax.ShapeDtypeStruct((T,), jnp.int32)),
    scratch_types=[
        pltpu.VMEM((_CHUNK,), jnp.int32),
        pltpu.VMEM((_CHUNK,), jnp.float32),
        pltpu.VMEM((_CHUNK,), jnp.int32),
        pltpu.VMEM((_CHUNK // 2,), jnp.int32),
        pltpu.VMEM((_CHUNK // 2,), jnp.int32),
        pltpu.VMEM((16,), jnp.int32),
        pltpu.VMEM((16 * _NSUB,), jnp.int32),
        pltpu.VMEM((_LSLICE,), jnp.float32),
        pltpu.VMEM((NBPAD,), jnp.int32),
        pltpu.VMEM_SHARED((16 * _NSUB,), jnp.int32),
        pltpu.VMEM_SHARED((L,), jnp.float32),
    ],
)
def _sc_sort(topi_hbm, topw_hbm, sw_hbm, be_hbm, inv0_hbm, inv1_hbm,
             ti_v, tw_v, pos_v, pe_v, po_v, cnt_v, grid_v, zf_v, be_v,
             cnt_sh, sw_sh):
    cid = lax.axis_index("c")
    sid = lax.axis_index("s")

    @pl.when(cid == 0)
    def _():
        base_j = sid * _CHUNK
        lanes = _lane_iota()
        zvec = jnp.zeros((16,), jnp.int32)
        pltpu.sync_copy(topi_hbm.at[pl.ds(base_j, _CHUNK)], ti_v)
        pltpu.sync_copy(topw_hbm.at[pl.ds(base_j, _CHUNK)], tw_v)

        cvecs = [jnp.zeros((16,), jnp.int32) for _ in range(NUM_EXPERTS)]
        for v in range(_CHUNK // 16):
            ev = ti_v[pl.ds(v * 16, 16)]
            for e in range(NUM_EXPERTS):
                cvecs[e] = cvecs[e] + jnp.where(ev == e, 1, 0)
        counts = zvec
        for e in range(NUM_EXPERTS):
            c = cvecs[e]
            s = c[0]
            for i in range(1, 16):
                s = s + c[i]
            counts = jnp.where(lanes == e, zvec + s, counts)
        cnt_v[...] = counts
        pltpu.sync_copy(cnt_v, cnt_sh.at[pl.ds(sid * 16, 16)])

        for v in range(_LSLICE // 16):
            zf_v[pl.ds(v * 16, 16)] = jnp.zeros((16,), jnp.float32)
        pltpu.sync_copy(zf_v, sw_sh.at[pl.ds(sid * _LSLICE, _LSLICE)])
        plsc.subcore_barrier()

        pltpu.sync_copy(cnt_sh, grid_v)
        tot = [jnp.int32(0)] * NUM_EXPERTS
        soff = [jnp.int32(0)] * NUM_EXPERTS
        for r in range(_NSUB):
            row = grid_v[pl.ds(r * 16, 16)]
            mr = jnp.where(r < sid, 1, 0)
            for e in range(NUM_EXPERTS):
                re = row[e]
                tot[e] = tot[e] + re
                soff[e] = soff[e] + re * mr
        pad = [jnp.bitwise_and(t + (BLK - 1), -BLK) for t in tot]
        base = [0] * NUM_EXPERTS
        acc = pad[0] * 0
        for e in range(NUM_EXPERTS):
            base[e] = acc
            acc = acc + pad[e]
        start = [base[e] + soff[e] for e in range(NUM_EXPERTS)]

        @pl.when(sid == 0)
        def _():
            es = [zvec] * (NBPAD // 16)
            for e in range(NUM_EXPERTS):
                endb = zvec + (base[e] + pad[e]) // BLK
                for g in range(NBPAD // 16):
                    es[g] = es[g] + jnp.where(lanes + g * 16 >= endb, 1, 0)
            for g in range(NBPAD // 16):
                be_v[pl.ds(g * 16, 16)] = jnp.minimum(es[g], NUM_EXPERTS - 1)
            pltpu.sync_copy(be_v, be_hbm)

        def step(v, runs):
            ev = ti_v[pl.ds(v * 16, 16)]
            pos = zvec
            for i in range(16):
                e = ev[i]
                ms = [1 - jnp.minimum(jnp.abs(e - k), 1)
                      for k in range(NUM_EXPERTS)]
                p = runs[0] * ms[0]
                for k in range(1, NUM_EXPERTS):
                    p = p + runs[k] * ms[k]
                pos = jnp.where(lanes == i, zvec + p, pos)
                runs = tuple(runs[k] + ms[k] for k in range(NUM_EXPERTS))
            pos_v[pl.ds(v * 16, 16)] = pos
            return runs

        lax.fori_loop(0, _CHUNK // 16, step,
                      tuple(jnp.int32(0) + s for s in start))
        for g in range(_CHUNK // 32):
            v0 = pos_v[pl.ds(g * 32, 16)]
            v1 = pos_v[pl.ds(g * 32 + 16, 16)]
            pe = zvec
            po = zvec
            for i in range(8):
                pe = jnp.where(lanes == i, zvec + v0[2 * i], pe)
                pe = jnp.where(lanes == i + 8, zvec + v1[2 * i], pe)
                po = jnp.where(lanes == i, zvec + v0[2 * i + 1], po)
                po = jnp.where(lanes == i + 8, zvec + v1[2 * i + 1], po)
            pe_v[pl.ds(g * 16, 16)] = pe
            po_v[pl.ds(g * 16, 16)] = po
        tbase = sid * (_CHUNK // 2)
        pltpu.sync_copy(pe_v, inv0_hbm.at[pl.ds(tbase, _CHUNK // 2)])
        pltpu.sync_copy(po_v, inv1_hbm.at[pl.ds(tbase, _CHUNK // 2)])
        pltpu.sync_copy(tw_v, sw_sh.at[pos_v], add=True)
        plsc.subcore_barrier()

        pltpu.sync_copy(sw_sh.at[pl.ds(sid * _LSLICE, _LSLICE)], zf_v)
        pltpu.sync_copy(zf_v, sw_hbm.at[pl.ds(sid * _LSLICE, _LSLICE)])


_GMESH = plsc.VectorSubcoreMesh(core_axis_name="c", subcore_axis_name="s")
_NW = 32
_SROWS = T // _NW


@functools.partial(
    pl.kernel, mesh=_GMESH,
    out_type=jax.ShapeDtypeStruct((L, HIDDEN), jnp.float32),
    scratch_types=[
        pltpu.VMEM((_SROWS,), jnp.int32),
        pltpu.VMEM((_SROWS,), jnp.int32),
        pltpu.VMEM((_SROWS, HIDDEN), jnp.float32),
        pltpu.SemaphoreType.DMA,
        pltpu.SemaphoreType.DMA,
    ],
)
def _sc_scatter(x_hbm, inv0_hbm, inv1_hbm, xs_hbm, i0_v, i1_v, xr_v,
                sem0, sem1):
    wid = lax.axis_index("s") * 2 + lax.axis_index("c")
    base = wid * _SROWS
    pltpu.sync_copy(inv0_hbm.at[pl.ds(base, _SROWS)], i0_v)
    pltpu.sync_copy(inv1_hbm.at[pl.ds(base, _SROWS)], i1_v)
    pltpu.sync_copy(x_hbm.at[pl.ds(base, _SROWS)], xr_v)
    cp0 = pltpu.async_copy(xr_v, xs_hbm.at[i0_v], sem0)
    cp1 = pltpu.async_copy(xr_v, xs_hbm.at[i1_v], sem1)
    cp0.wait()
    cp1.wait()


def _experts_body(be_ref, xs_ref, gup_ref, dnt_ref, sw_ref, out_ref):
    xb = xs_ref[...].astype(jnp.bfloat16)
    gu = lax.dot_general(
        xb, gup_ref[0], (((1,), (0,)), ((), ())),
        preferred_element_type=jnp.float32)
    g = gu[:, :INTER]
    u = gu[:, INTER:]
    h = (g * (1.0 / (1.0 + jnp.exp(-g))) * u).astype(jnp.bfloat16)
    oe = lax.dot_general(
        h, dnt_ref[0], (((1,), (0,)), ((), ())),
        preferred_element_type=jnp.float32)
    out_ref[...] = oe * sw_ref[...]


def _experts(be, xs, gup_t, dnt, sw):
    grid_spec = pltpu.PrefetchScalarGridSpec(
        num_scalar_prefetch=1,
        grid=(NB,),
        in_specs=[
            pl.BlockSpec((BLK, HIDDEN), lambda b, be_ref: (b, 0)),
            pl.BlockSpec((1, HIDDEN, 2 * INTER),
                         lambda b, be_ref: (be_ref[b], 0, 0)),
            pl.BlockSpec((1, INTER, HIDDEN),
                         lambda b, be_ref: (be_ref[b], 0, 0)),
            pl.BlockSpec((BLK, 1), lambda b, be_ref: (b, 0)),
        ],
        out_specs=pl.BlockSpec((BLK, HIDDEN), lambda b, be_ref: (b, 0)),
    )
    return pl.pallas_call(
        _experts_body,
        grid_spec=grid_spec,
        out_shape=jax.ShapeDtypeStruct((L, HIDDEN), jnp.float32),
    )(be, xs, gup_t, dnt, sw)


_CTOK = T // _NW
_CCH = 16


@functools.partial(
    pl.kernel, mesh=_GMESH,
    out_type=jax.ShapeDtypeStruct((T, HIDDEN), jnp.float32),
    scratch_types=[
        pltpu.VMEM((_CTOK,), jnp.int32),
        pltpu.VMEM((_CTOK,), jnp.int32),
        pltpu.VMEM((_CCH, HIDDEN), jnp.float32),
        pltpu.VMEM((_CCH, HIDDEN), jnp.float32),
        pltpu.VMEM((_CCH, HIDDEN), jnp.float32),
        pltpu.SemaphoreType.DMA,
        pltpu.SemaphoreType.DMA,
    ],
)
def _sc_combine(os_hbm, inv0_hbm, inv1_hbm, out_hbm, i0_v, i1_v,
                g0_v, g1_v, o_v, sem0, sem1):
    wid = lax.axis_index("s") * 2 + lax.axis_index("c")
    tbase = wid * _CTOK
    pltpu.sync_copy(inv0_hbm.at[pl.ds(tbase, _CTOK)], i0_v)
    pltpu.sync_copy(inv1_hbm.at[pl.ds(tbase, _CTOK)], i1_v)
    for c in range(_CTOK // _CCH):
        cp0 = pltpu.async_copy(
            os_hbm.at[i0_v.at[pl.ds(c * _CCH, _CCH)]], g0_v, sem0)
        cp1 = pltpu.async_copy(
            os_hbm.at[i1_v.at[pl.ds(c * _CCH, _CCH)]], g1_v, sem1)
        cp0.wait()
        cp1.wait()

        def body(q, _):
            for i in range(_CCH):
                a = g0_v[i, pl.ds(q * 16, 16)]
                b = g1_v[i, pl.ds(q * 16, 16)]
                o_v[i, pl.ds(q * 16, 16)] = a + b
            return 0

        lax.fori_loop(0, HIDDEN // 16, body, 0)
        pltpu.sync_copy(o_v, out_hbm.at[pl.ds(tbase + c * _CCH, _CCH)])


def kernel(hidden_states, gate_weight, gate_up_proj, down_proj):
    Bb, Ss, H = hidden_states.shape
    x = hidden_states.reshape(T, H)
    gwt = gate_weight.T
    gup_t = gate_up_proj.transpose(0, 2, 1).astype(jnp.bfloat16)
    dnt = down_proj.transpose(0, 2, 1).astype(jnp.bfloat16)

    topi, topw = _router(x, gwt)
    topi_flat = topi.reshape(A)
    topw_flat = topw.reshape(A)
    sw, be, inv0, inv1 = _sc_sort(topi_flat, topw_flat)
    xs = _sc_scatter(x, inv0, inv1)
    out_sorted = _experts(be, xs, gup_t, dnt, sw.reshape(L, 1))
    out = _sc_combine(out_sorted, inv0, inv1)
    return out.reshape(Bb, Ss, H)
